# Initial kernel scaffold; baseline (speedup 1.0000x reference)
import jax
import jax.numpy as jnp
from jax.experimental import pallas as pl


def kernel(A, B, predicted_A, predicted_B, file):
    raise NotImplementedError("write your pallas kernel here")


# speedup vs baseline: 1.0604x; 1.0604x over previous
import jax
import jax.numpy as jnp
from jax import lax
from jax.experimental import pallas as pl
from jax.experimental.pallas import tpu as pltpu

_N = 20480
_TQ = 256
_TK = 2048
_QT = _N // _TQ
_KT = _N // _TK
_INF = float("inf")


def _bounds_scalars(bounds_ref, m):
    return (bounds_ref[m, 0], bounds_ref[m, 1], bounds_ref[m, 2],
            bounds_ref[m, 3], bounds_ref[m, 4])


def _mask(x, y, z, b):
    xlo, xhi, yhi, zhi, zlo = b
    return (x >= xlo) & (x <= xhi) & (y <= yhi) & (z <= zhi) & (z >= zlo)


def _chamfer_body(bounds_ref, PA, PB, A3, B3, AT, BT, out_ref,
                  colmin_ref, acc_ref):
    p = pl.program_id(0)
    qt = pl.program_id(1)
    is_ab = p < 2
    mq = jnp.where(is_ab, p + 2, p - 2)
    qb = _bounds_scalars(bounds_ref, mq)
    kb = _bounds_scalars(bounds_ref, p)

    qsl = pl.ds(qt * _TQ, _TQ)
    Qt = jnp.where(is_ab, PA[qsl, :], PB[qsl, :])
    Mt = jnp.where(is_ab, B3[qsl, :], A3[qsl, :])
    qn = jnp.sum(Qt * Qt, axis=1, keepdims=True)
    Qaug = jnp.concatenate(
        [Qt, qn, jnp.ones((_TQ, 1), jnp.float32)], axis=1)
    qmask = _mask(Mt[:, 0:1], Mt[:, 1:2], Mt[:, 2:3], qb)

    @pl.when(qt == 0)
    def _():
        acc_ref[0] = 0.0
        acc_ref[1] = 0.0

    @pl.when((p == 0) & (qt == 0))
    def _():
        acc_ref[4] = 0.0

    def kt_body(kt, rowmin):
        ksl = pl.ds(kt * _TK, _TK)
        Kt3 = jnp.where(is_ab, AT[:, ksl], BT[:, ksl])
        kx, ky, kz = Kt3[0:1, :], Kt3[1:2, :], Kt3[2:3, :]
        kn = kx * kx + ky * ky + kz * kz
        Kaug = jnp.concatenate(
            [-2.0 * Kt3, jnp.ones((1, _TK), jnp.float32), kn], axis=0)
        D = lax.dot_general(Qaug, Kaug, (((1,), (0,)), ((), ())),
                            preferred_element_type=jnp.float32)
        D = jnp.maximum(D, 1e-12)
        kmask = _mask(kx, ky, kz, kb)
        rm = jnp.min(jnp.where(kmask, D, _INF), axis=1, keepdims=True)
        cm = jnp.min(jnp.where(qmask, D, _INF), axis=0, keepdims=True)
        prev = jnp.where(qt == 0, jnp.full((1, _TK), _INF),
                         colmin_ref[0:1, ksl])
        colmin_ref[0:1, ksl] = jnp.minimum(prev, cm)
        return jnp.minimum(rowmin, rm)

    rowmin = lax.fori_loop(0, _KT, kt_body,
                           jnp.full((_TQ, 1), _INF))

    acc_ref[0] = acc_ref[0] + jnp.sum(
        jnp.where(qmask, jnp.sqrt(rowmin), 0.0))
    acc_ref[1] = acc_ref[1] + jnp.sum(qmask.astype(jnp.float32))

    @pl.when(qt == _QT - 1)
    def _():
        def col_body(kt, cs_cc):
            cs, cc = cs_cc
            ksl = pl.ds(kt * _TK, _TK)
            Kt3 = jnp.where(is_ab, AT[:, ksl], BT[:, ksl])
            kmask = _mask(Kt3[0:1, :], Kt3[1:2, :], Kt3[2:3, :], kb)
            cm = colmin_ref[0:1, ksl]
            cs = cs + jnp.sum(jnp.where(kmask, jnp.sqrt(cm), 0.0))
            cc = cc + jnp.sum(kmask.astype(jnp.float32))
            return cs, cc

        cs, cc = lax.fori_loop(0, _KT, col_body,
                               (jnp.float32(0.0), jnp.float32(0.0)))
        pair_loss = 0.5 * (acc_ref[0] / acc_ref[1] + cs / cc)
        acc_ref[4] = acc_ref[4] + pair_loss

        @pl.when(p == 3)
        def _():
            out_ref[0, 0] = acc_ref[4]


def _chamfer_call(bounds, PA, PB, A3, B3, AT, BT, interpret=False):
    full = lambda shape: pl.BlockSpec(shape, lambda p, q: (0,) * len(shape))
    return pl.pallas_call(
        _chamfer_body,
        grid=(4, _QT),
        in_specs=[
            pl.BlockSpec(memory_space=pltpu.SMEM),
            full((_N, 3)), full((_N, 3)), full((_N, 3)), full((_N, 3)),
            full((3, _N)), full((3, _N)),
        ],
        out_specs=pl.BlockSpec(memory_space=pltpu.SMEM),
        out_shape=jax.ShapeDtypeStruct((1, 1), jnp.float32),
        scratch_shapes=[
            pltpu.VMEM((1, _N), jnp.float32),
            pltpu.SMEM((8,), jnp.float32),
        ],
        compiler_params=pltpu.CompilerParams(
            dimension_semantics=("arbitrary", "arbitrary")),
        interpret=interpret,
    )(bounds, PA, PB, A3, B3, AT, BT)


def kernel(A, B, predicted_A, predicted_B, file):
    A3, B3 = A[0], B[0]
    PA, PB = predicted_A[0], predicted_B[0]
    AT, BT = A3.T, B3.T
    inf = jnp.float32(jnp.inf)
    y1 = jnp.minimum(file[1, 1], file[1, 4])
    y3 = jnp.minimum(file[3, 1], file[3, 4])
    bounds = jnp.stack([
        jnp.stack([file[0, 0], file[0, 3], inf, file[0, 2], file[0, 5]]),
        jnp.stack([file[1, 0], file[1, 3], y1, file[1, 2], file[1, 5]]),
        jnp.stack([file[2, 0], file[2, 3], inf, file[2, 2], file[2, 5]]),
        jnp.stack([file[1, 0], file[3, 3], y3, file[3, 2], file[3, 5]]),
    ])
    out = _chamfer_call(bounds, PA, PB, A3, B3, AT, BT)
    return out[0, 0]
